# baseline (speedup 1.0000x reference)
import jax
import jax.numpy as jnp
from jax.experimental import pallas as pl
from jax.experimental.pallas import tpu as pltpu


_LANES = 256
_LCHUNK = 40


def _add_body(x_ref, t_ref, o_ref, tb_ref):
    @pl.when(pl.program_id(1) == 0)
    def _():
        tb_ref[...] = jax.lax.broadcast_in_dim(
            t_ref[...], tb_ref.shape, (0, 1)
        )

    o_ref[...] = x_ref[...] + tb_ref[...]


def kernel(x, pos_table):
    B, L, D = x.shape
    xt = x.transpose(1, 2, 0)
    out_t = pl.pallas_call(
        _add_body,
        grid=(L // _LCHUNK, B // _LANES),
        in_specs=[
            pl.BlockSpec((_LCHUNK, D, _LANES), lambda i, j: (i, 0, j)),
            pl.BlockSpec((_LCHUNK, D), lambda i, j: (i, 0)),
        ],
        out_specs=pl.BlockSpec((_LCHUNK, D, _LANES), lambda i, j: (i, 0, j)),
        out_shape=jax.ShapeDtypeStruct((L, D, B), x.dtype),
        scratch_shapes=[pltpu.VMEM((_LCHUNK, D, _LANES), x.dtype)],
        compiler_params=pltpu.CompilerParams(
            dimension_semantics=("arbitrary", "arbitrary"),
        ),
    )(xt, pos_table)
    return out_t.transpose(2, 0, 1)


# speedup vs baseline: 1.0786x; 1.0786x over previous
import jax
import jax.numpy as jnp
from jax.experimental import pallas as pl
from jax.experimental.pallas import tpu as pltpu


_LCHUNK = 8


def _add_body(x_ref, t_ref, o_ref):
    o_ref[...] = x_ref[...] + jax.lax.broadcast_in_dim(
        t_ref[...], o_ref.shape, (0, 1)
    )


def kernel(x, pos_table):
    B, L, D = x.shape
    xt = x.transpose(1, 2, 0)
    out_t = pl.pallas_call(
        _add_body,
        grid=(L // _LCHUNK,),
        in_specs=[
            pl.BlockSpec((_LCHUNK, D, B), lambda i: (i, 0, 0)),
            pl.BlockSpec((_LCHUNK, D), lambda i: (i, 0)),
        ],
        out_specs=pl.BlockSpec((_LCHUNK, D, B), lambda i: (i, 0, 0)),
        out_shape=jax.ShapeDtypeStruct((L, D, B), x.dtype),
        compiler_params=pltpu.CompilerParams(
            dimension_semantics=("arbitrary",),
        ),
    )(xt, pos_table)
    return out_t.transpose(2, 0, 1)
